# Initial kernel scaffold; baseline (speedup 1.0000x reference)
import jax
import jax.numpy as jnp
from jax.experimental import pallas as pl


def kernel(x, W1, b1, W2, b2, keys_mem, values_mem):
    raise NotImplementedError("write your pallas kernel here")


# speedup vs baseline: 14.6054x; 14.6054x over previous
import functools

import jax
import jax.numpy as jnp
from jax.experimental import pallas as pl
from jax.experimental.pallas import tpu as pltpu

KNN = 50
EPS = 1e-3
I32MIN = jnp.iinfo(jnp.int32).min
I32MAX = jnp.iinfo(jnp.int32).max


def _fp_to_key(d2):
    b = jax.lax.bitcast_convert_type(d2, jnp.int32)
    return jnp.where(b < 0, b ^ jnp.int32(0x7FFFFFFF), b)


def _key_to_fp(m):
    b = jnp.where(m < 0, m ^ jnp.int32(0x7FFFFFFF), m)
    return jax.lax.bitcast_convert_type(b, jnp.float32)


def _floor_avg(lo, hi):
    return (lo >> 1) + (hi >> 1) + (lo & hi & 1)


def _nec_kernel(x_ref, w1_ref, b1_ref, w2_ref, b2_ref, keys_ref, ksq_ref,
                vals_ref, outv_ref, outa_ref, z_s, m_s, *, B, A, CAP, QB, KT):
    a = pl.program_id(0)
    qb = pl.program_id(1)
    NQB = B // QB

    def _rb(v):
        return v.astype(jnp.bfloat16).astype(jnp.float32)

    @pl.when((a == 0) & (qb == 0))
    def _encode():
        xv = _rb(x_ref[...])
        w1 = _rb(w1_ref[...])
        h = xv[:, 0:1] * w1[0:1, :] + xv[:, 1:2] * w1[1:2, :]
        h = jnp.maximum(h + b1_ref[...], 0.0)
        hb = _rb(h)
        w2 = _rb(w2_ref[...])
        z = hb[:, 0:1] * w2[0:1, :]
        for j in range(1, hb.shape[1]):
            z = z + hb[:, j:j + 1] * w2[j:j + 1, :]
        z_s[...] = z + b2_ref[...]

    rows = pl.ds(qb * QB, QB)
    z_blk = z_s[rows, :]
    zb = z_blk.astype(jnp.bfloat16)
    qsq = jnp.sum(z_blk * z_blk, axis=1, keepdims=True)

    rmin = jnp.full((QB, 1), I32MAX, jnp.int32)
    rmax = jnp.full((QB, 1), I32MIN, jnp.int32)
    for kt in range(CAP // KT):
        ks = keys_ref[0, :, pl.ds(kt * KT, KT)]
        dots = jax.lax.dot_general(
            zb, ks, (((1,), (0,)), ((), ())),
            preferred_element_type=jnp.float32)
        d2 = (qsq + ksq_ref[0, :, pl.ds(kt * KT, KT)]) - 2.0 * dots
        m = _fp_to_key(d2)
        m_s[:, pl.ds(kt * KT, KT)] = m
        rmin = jnp.minimum(rmin, jnp.min(m, axis=1, keepdims=True))
        rmax = jnp.maximum(rmax, jnp.max(m, axis=1, keepdims=True))

    NT = CAP // KT
    zero = jnp.float32(0.0)

    def _bis_cond(c):
        lo, hi = c
        return jnp.any(lo < hi)

    def _bis_body(c):
        lo, hi = c
        mid = _floor_avg(lo, hi)
        cnt = jnp.zeros((QB, 1), jnp.int32)
        mx = jnp.full((QB, 1), I32MIN, jnp.int32)
        mn = jnp.full((QB, 1), I32MAX, jnp.int32)
        for kt in range(NT):
            mmt = m_s[:, pl.ds(kt * KT, KT)]
            le = mmt <= mid
            cnt += jnp.sum(le.astype(jnp.int32), axis=1, keepdims=True)
            mx = jnp.maximum(mx, jnp.max(jnp.where(le, mmt, I32MIN), axis=1,
                                         keepdims=True))
            mn = jnp.minimum(mn, jnp.min(jnp.where(le, I32MAX, mmt), axis=1,
                                         keepdims=True))
        ge = cnt >= KNN
        return jnp.where(ge, lo, mn), jnp.where(ge, mx, hi)

    lo, hi = jax.lax.while_loop(_bis_cond, _bis_body, (rmin, rmax))
    t = hi

    cnt_lt = jnp.zeros((QB, 1), jnp.int32)
    cnt_eq = jnp.zeros((QB, 1), jnp.int32)
    s_w = jnp.zeros((QB, 1), jnp.float32)
    s_wv = jnp.zeros((QB, 1), jnp.float32)
    for kt in range(NT):
        mmt = m_s[:, pl.ds(kt * KT, KT)]
        vrow = vals_ref[0, :, pl.ds(kt * KT, KT)]
        w = 1.0 / (_key_to_fp(mmt) + EPS)
        lt = mmt < t
        cnt_lt += jnp.sum(lt.astype(jnp.int32), axis=1, keepdims=True)
        cnt_eq += jnp.sum((mmt == t).astype(jnp.int32), axis=1, keepdims=True)
        s_w += jnp.sum(jnp.where(lt, w, zero), axis=1, keepdims=True)
        s_wv += jnp.sum(jnp.where(lt, w * vrow, zero), axis=1, keepdims=True)
    need = KNN - cnt_lt

    def _idx_cond(c):
        lo_i, hi_i = c
        return jnp.any(lo_i + 1 < hi_i)

    def _idx_body(c):
        lo_i, hi_i = c
        mid = _floor_avg(lo_i, hi_i)
        csel = jnp.zeros((QB, 1), jnp.int32)
        for kt in range(NT):
            mmt = m_s[:, pl.ds(kt * KT, KT)]
            iot = jax.lax.broadcasted_iota(jnp.int32, (QB, KT), 1) + kt * KT
            sel = (mmt == t) & (iot <= mid)
            csel += jnp.sum(sel.astype(jnp.int32), axis=1, keepdims=True)
        ge = csel >= need
        return jnp.where(ge, lo_i, mid), jnp.where(ge, mid, hi_i)

    lo_i0 = jnp.where(cnt_eq == need, jnp.int32(CAP - 2), jnp.int32(-1))
    hi_i0 = jnp.full((QB, 1), CAP - 1, jnp.int32)
    _, hi_i = jax.lax.while_loop(_idx_cond, _idx_body, (lo_i0, hi_i0))

    sv_eq = jnp.zeros((QB, 1), jnp.float32)
    for kt in range(NT):
        mmt = m_s[:, pl.ds(kt * KT, KT)]
        vrow = vals_ref[0, :, pl.ds(kt * KT, KT)]
        iot = jax.lax.broadcasted_iota(jnp.int32, (QB, KT), 1) + kt * KT
        sel = (mmt == t) & (iot <= hi_i)
        sv_eq += jnp.sum(jnp.where(sel, vrow, zero), axis=1, keepdims=True)
    wt = 1.0 / (_key_to_fp(t) + EPS)
    vals_col = (s_wv + wt * sv_eq) / (s_w + need.astype(jnp.float32) * wt)

    colidx = jax.lax.broadcasted_iota(jnp.int32, (QB, A), 1)
    cur = outv_ref[rows, :]
    outv_ref[rows, :] = jnp.where(colidx == a, vals_col, cur)

    @pl.when((a == A - 1) & (qb == NQB - 1))
    def _argmax():
        vall = outv_ref[...]
        best = vall[:, 0:1]
        arg = jnp.zeros((B, 1), jnp.int32)
        for j in range(1, A):
            vj = vall[:, j:j + 1]
            gt = vj > best
            arg = jnp.where(gt, jnp.int32(j), arg)
            best = jnp.where(gt, vj, best)
        outa_ref[...] = arg


def kernel(x, W1, b1, W2, b2, keys_mem, values_mem):
    B, IN = x.shape
    H = W1.shape[1]
    D = W2.shape[1]
    A, CAP, _ = keys_mem.shape
    QB = min(128, B)
    KT = min(2048, CAP)

    ksq = jnp.sum(keys_mem * keys_mem, axis=-1).reshape(A, 1, CAP)
    keys_t = jnp.swapaxes(keys_mem, 1, 2).astype(jnp.bfloat16)
    vals3 = values_mem.reshape(A, 1, CAP)
    b1r = b1.reshape(1, H)
    b2r = b2.reshape(1, D)

    grid = (A, B // QB)
    kern = functools.partial(_nec_kernel, B=B, A=A, CAP=CAP, QB=QB, KT=KT)
    values, actions = pl.pallas_call(
        kern,
        grid=grid,
        in_specs=[
            pl.BlockSpec((B, IN), lambda a, q: (0, 0)),
            pl.BlockSpec((IN, H), lambda a, q: (0, 0)),
            pl.BlockSpec((1, H), lambda a, q: (0, 0)),
            pl.BlockSpec((H, D), lambda a, q: (0, 0)),
            pl.BlockSpec((1, D), lambda a, q: (0, 0)),
            pl.BlockSpec((1, D, CAP), lambda a, q: (a, 0, 0)),
            pl.BlockSpec((1, 1, CAP), lambda a, q: (a, 0, 0)),
            pl.BlockSpec((1, 1, CAP), lambda a, q: (a, 0, 0)),
        ],
        out_specs=[
            pl.BlockSpec((B, A), lambda a, q: (0, 0)),
            pl.BlockSpec((B, 1), lambda a, q: (0, 0)),
        ],
        out_shape=[
            jax.ShapeDtypeStruct((B, A), jnp.float32),
            jax.ShapeDtypeStruct((B, 1), jnp.int32),
        ],
        scratch_shapes=[
            pltpu.VMEM((B, D), jnp.float32),
            pltpu.VMEM((QB, CAP), jnp.int32),
        ],
        compiler_params=pltpu.CompilerParams(
            dimension_semantics=("arbitrary", "arbitrary"),
        ),
    )(x, W1, b1r, W2, b2r, keys_t, ksq, vals3)
    return values, actions.reshape(B)


# speedup vs baseline: 17.5368x; 1.2007x over previous
import functools

import jax
import jax.numpy as jnp
from jax.experimental import pallas as pl
from jax.experimental.pallas import tpu as pltpu

KNN = 50
EPS = 1e-3
I32MIN = jnp.iinfo(jnp.int32).min
I32MAX = jnp.iinfo(jnp.int32).max


def _fp_to_key(d2):
    b = jax.lax.bitcast_convert_type(d2, jnp.int32)
    return jnp.where(b < 0, b ^ jnp.int32(0x7FFFFFFF), b)


def _key_to_fp(m):
    b = jnp.where(m < 0, m ^ jnp.int32(0x7FFFFFFF), m)
    return jax.lax.bitcast_convert_type(b, jnp.float32)


def _floor_avg(lo, hi):
    return (lo >> 1) + (hi >> 1) + (lo & hi & 1)


def _nec_kernel(x_ref, w1_ref, b1_ref, w2_ref, b2_ref, keys_ref, ksq_ref,
                vals_ref, outv_ref, outa_ref, z_s, m_s, cm_s,
                *, B, A, CAP, QB, KT):
    a = pl.program_id(0)
    qb = pl.program_id(1)
    NQB = B // QB

    def _rb(v):
        return v.astype(jnp.bfloat16).astype(jnp.float32)

    @pl.when((a == 0) & (qb == 0))
    def _encode():
        xv = _rb(x_ref[...])
        w1 = _rb(w1_ref[...])
        h = xv[:, 0:1] * w1[0:1, :] + xv[:, 1:2] * w1[1:2, :]
        h = jnp.maximum(h + b1_ref[...], 0.0)
        hb = _rb(h)
        w2 = _rb(w2_ref[...])
        z = hb[:, 0:1] * w2[0:1, :]
        for j in range(1, hb.shape[1]):
            z = z + hb[:, j:j + 1] * w2[j:j + 1, :]
        z_s[...] = z + b2_ref[...]

    rows = pl.ds(qb * QB, QB)
    z_blk = z_s[rows, :]
    zb = z_blk.astype(jnp.bfloat16)
    qsq = jnp.sum(z_blk * z_blk, axis=1, keepdims=True)

    NT = CAP // KT
    NCH = KT // 16
    zero = jnp.float32(0.0)
    for kt in range(NT):
        ks = keys_ref[0, :, pl.ds(kt * KT, KT)]
        dots = jax.lax.dot_general(
            zb, ks, (((1,), (0,)), ((), ())),
            preferred_element_type=jnp.float32)
        d2 = (qsq + ksq_ref[0, :, pl.ds(kt * KT, KT)]) - 2.0 * dots
        m = _fp_to_key(d2)
        m_s[:, pl.ds(kt * KT, KT)] = m
        cm = m
        while cm.shape[1] > NCH:
            half = cm.shape[1] // 2
            cm = jnp.minimum(cm[:, :half], cm[:, half:])
        cm_s[:, pl.ds(kt * NCH, NCH)] = cm

    def _rank_bisect(loaders, lo, hi):
        def cond(c):
            lo, hi = c
            return jnp.any(lo < hi)

        def body(c):
            lo, hi = c
            mid = _floor_avg(lo, hi)
            cnt = jnp.zeros((QB, 1), jnp.int32)
            mx = jnp.full((QB, 1), I32MIN, jnp.int32)
            mn = jnp.full((QB, 1), I32MAX, jnp.int32)
            for ld in loaders:
                mmt = ld()
                le = mmt <= mid
                cnt += jnp.sum(le.astype(jnp.int32), axis=1, keepdims=True)
                mx = jnp.maximum(mx, jnp.max(jnp.where(le, mmt, I32MIN),
                                             axis=1, keepdims=True))
                mn = jnp.minimum(mn, jnp.min(jnp.where(le, I32MAX, mmt),
                                             axis=1, keepdims=True))
            ge = cnt >= KNN
            return jnp.where(ge, lo, mn), jnp.where(ge, mx, hi)

        return jax.lax.while_loop(cond, body, (lo, hi))[1]

    cm_all = cm_s[...]
    rmin = jnp.min(cm_all, axis=1, keepdims=True)
    cmax = jnp.max(cm_all, axis=1, keepdims=True)
    t_cm = _rank_bisect([lambda: cm_s[...]], rmin, cmax)
    t = _rank_bisect(
        [(lambda kt=kt: m_s[:, pl.ds(kt * KT, KT)]) for kt in range(NT)],
        rmin, t_cm)

    cnt_lt = jnp.zeros((QB, 1), jnp.int32)
    cnt_eq = jnp.zeros((QB, 1), jnp.int32)
    s_w = jnp.zeros((QB, 1), jnp.float32)
    s_wv = jnp.zeros((QB, 1), jnp.float32)
    sv_eq_all = jnp.zeros((QB, 1), jnp.float32)
    for kt in range(NT):
        mmt = m_s[:, pl.ds(kt * KT, KT)]
        vrow = vals_ref[0, :, pl.ds(kt * KT, KT)]
        w = 1.0 / (_key_to_fp(mmt) + EPS)
        lt = mmt < t
        eq = mmt == t
        cnt_lt += jnp.sum(lt.astype(jnp.int32), axis=1, keepdims=True)
        cnt_eq += jnp.sum(eq.astype(jnp.int32), axis=1, keepdims=True)
        s_w += jnp.sum(jnp.where(lt, w, zero), axis=1, keepdims=True)
        s_wv += jnp.sum(jnp.where(lt, w * vrow, zero), axis=1, keepdims=True)
        sv_eq_all += jnp.sum(jnp.where(eq, vrow, zero), axis=1,
                             keepdims=True)
    need = KNN - cnt_lt

    def _ties_resolved():
        def _idx_cond(c):
            lo_i, hi_i = c
            return jnp.any(lo_i + 1 < hi_i)

        def _idx_body(c):
            lo_i, hi_i = c
            mid = _floor_avg(lo_i, hi_i)
            csel = jnp.zeros((QB, 1), jnp.int32)
            for kt in range(NT):
                mmt = m_s[:, pl.ds(kt * KT, KT)]
                iot = (jax.lax.broadcasted_iota(jnp.int32, (QB, KT), 1)
                       + kt * KT)
                sel = (mmt == t) & (iot <= mid)
                csel += jnp.sum(sel.astype(jnp.int32), axis=1, keepdims=True)
            ge = csel >= need
            return jnp.where(ge, lo_i, mid), jnp.where(ge, mid, hi_i)

        lo_i0 = jnp.where(cnt_eq == need, jnp.int32(CAP - 2), jnp.int32(-1))
        hi_i0 = jnp.full((QB, 1), CAP - 1, jnp.int32)
        _, hi_i = jax.lax.while_loop(_idx_cond, _idx_body, (lo_i0, hi_i0))

        sv = jnp.zeros((QB, 1), jnp.float32)
        for kt in range(NT):
            mmt = m_s[:, pl.ds(kt * KT, KT)]
            vrow = vals_ref[0, :, pl.ds(kt * KT, KT)]
            iot = jax.lax.broadcasted_iota(jnp.int32, (QB, KT), 1) + kt * KT
            sel = (mmt == t) & (iot <= hi_i)
            sv += jnp.sum(jnp.where(sel, vrow, zero), axis=1, keepdims=True)
        return sv

    sv_eq = jax.lax.cond(jnp.all(cnt_eq == need),
                         lambda: sv_eq_all, _ties_resolved)
    wt = 1.0 / (_key_to_fp(t) + EPS)
    vals_col = (s_wv + wt * sv_eq) / (s_w + need.astype(jnp.float32) * wt)

    colidx = jax.lax.broadcasted_iota(jnp.int32, (QB, A), 1)
    cur = outv_ref[rows, :]
    outv_ref[rows, :] = jnp.where(colidx == a, vals_col, cur)

    @pl.when((a == A - 1) & (qb == NQB - 1))
    def _argmax():
        vall = outv_ref[...]
        best = vall[:, 0:1]
        arg = jnp.zeros((B, 1), jnp.int32)
        for j in range(1, A):
            vj = vall[:, j:j + 1]
            gt = vj > best
            arg = jnp.where(gt, jnp.int32(j), arg)
            best = jnp.where(gt, vj, best)
        outa_ref[...] = arg


def kernel(x, W1, b1, W2, b2, keys_mem, values_mem):
    B, IN = x.shape
    H = W1.shape[1]
    D = W2.shape[1]
    A, CAP, _ = keys_mem.shape
    QB = min(128, B)
    KT = min(2048, CAP)

    ksq = jnp.sum(keys_mem * keys_mem, axis=-1).reshape(A, 1, CAP)
    keys_t = jnp.swapaxes(keys_mem, 1, 2).astype(jnp.bfloat16)
    vals3 = values_mem.reshape(A, 1, CAP)
    b1r = b1.reshape(1, H)
    b2r = b2.reshape(1, D)

    grid = (A, B // QB)
    kern = functools.partial(_nec_kernel, B=B, A=A, CAP=CAP, QB=QB, KT=KT)
    values, actions = pl.pallas_call(
        kern,
        grid=grid,
        in_specs=[
            pl.BlockSpec((B, IN), lambda a, q: (0, 0)),
            pl.BlockSpec((IN, H), lambda a, q: (0, 0)),
            pl.BlockSpec((1, H), lambda a, q: (0, 0)),
            pl.BlockSpec((H, D), lambda a, q: (0, 0)),
            pl.BlockSpec((1, D), lambda a, q: (0, 0)),
            pl.BlockSpec((1, D, CAP), lambda a, q: (a, 0, 0)),
            pl.BlockSpec((1, 1, CAP), lambda a, q: (a, 0, 0)),
            pl.BlockSpec((1, 1, CAP), lambda a, q: (a, 0, 0)),
        ],
        out_specs=[
            pl.BlockSpec((B, A), lambda a, q: (0, 0)),
            pl.BlockSpec((B, 1), lambda a, q: (0, 0)),
        ],
        out_shape=[
            jax.ShapeDtypeStruct((B, A), jnp.float32),
            jax.ShapeDtypeStruct((B, 1), jnp.int32),
        ],
        scratch_shapes=[
            pltpu.VMEM((B, D), jnp.float32),
            pltpu.VMEM((QB, CAP), jnp.int32),
            pltpu.VMEM((QB, CAP // 16), jnp.int32),
        ],
        compiler_params=pltpu.CompilerParams(
            dimension_semantics=("arbitrary", "arbitrary"),
        ),
    )(x, W1, b1r, W2, b2r, keys_t, ksq, vals3)
    return values, actions.reshape(B)
